# baseline (speedup 1.0000x reference)
import functools

import jax
import jax.numpy as jnp
from jax import lax
from jax.experimental import pallas as pl
from jax.experimental.pallas import tpu as pltpu
from jax.experimental.pallas import tpu_sc as plsc

N = 10000
E = 160000
D_IN = 256
HID = 256
OUT = 256
HEADS = 2

NPAD = 10240
EB = 128
NB0 = 24
NB1 = 56
NBMAX = max(NB0, NB1)
EPAD = 16 * (NB0 + NB1) * EB
EPAD_ARR = EPAD + (NBMAX - min(NB0, NB1)) * EB
ROWS_PT = NPAD // 16
RB = 256
GRID = NPAD // RB
CW = 64
VPR = CW // 16
NC1 = 2 * HID // CW
NC2 = OUT // CW


def _mm1_body(x_ref, w_ref, a_ref, xlc_ref, ab_ref, gmax_ref):
    i = pl.program_id(0)
    xl = jnp.dot(x_ref[...], w_ref[...], preferred_element_type=jnp.float32)
    for c in range(NC1):
        xlc_ref[c] = xl[:, c * CW:(c + 1) * CW]
    ab = jnp.dot(xl, a_ref[...], preferred_element_type=jnp.float32)
    ab_ref[...] = ab
    bmax = jnp.broadcast_to(jnp.max(ab, axis=0, keepdims=True), (8, 128))

    @pl.when(i == 0)
    def _():
        gmax_ref[...] = bmax

    @pl.when(i != 0)
    def _():
        gmax_ref[...] = jnp.maximum(gmax_ref[...], bmax)


def _mm2_body(p_ref, d_ref, b_ref, w_ref, a_ref, xlc_ref, ab_ref, gmax_ref):
    i = pl.program_id(0)
    d = d_ref[0] + d_ref[1]
    rden = 1.0 / (d + 1e-30)
    acc = jnp.zeros((RB, OUT), jnp.float32)
    for c in range(NC1):
        hh = c // (NC1 // 2)
        hc = (p_ref[0, c] + p_ref[1, c]) * rden[:, hh:hh + 1] + b_ref[c][None, :]
        hc = jnp.where(hc > 0.0, hc, jnp.exp(jnp.minimum(hc, 0.0)) - 1.0)
        acc = acc + jnp.dot(hc, w_ref[pl.ds(c * CW, CW), :],
                            preferred_element_type=jnp.float32)
    for c in range(NC2):
        xlc_ref[c] = acc[:, c * CW:(c + 1) * CW]
    ab = jnp.dot(acc, a_ref[...], preferred_element_type=jnp.float32)
    ab_ref[...] = ab
    bmax = jnp.broadcast_to(jnp.max(ab, axis=0, keepdims=True), (8, 128))

    @pl.when(i == 0)
    def _():
        gmax_ref[...] = bmax

    @pl.when(i != 0)
    def _():
        gmax_ref[...] = jnp.maximum(gmax_ref[...], bmax)


def _fin_body(p_ref, d_ref, b_ref, o_ref):
    d = d_ref[0] + d_ref[1]
    rden = 1.0 / (d + 1e-30)
    for c in range(NC2):
        pc = p_ref[0, c] + p_ref[1, c]
        o_ref[:, c * CW:(c + 1) * CW] = pc * rden[:, 0:1] + b_ref[c][None, :]


def _make_sc_edge(heads, nchunks):
    cph = nchunks // heads
    mesh = plsc.VectorSubcoreMesh(core_axis_name="core", subcore_axis_name="subcore",
                                  num_cores=2, num_subcores=16)

    @functools.partial(
        pl.kernel,
        compiler_params=pltpu.CompilerParams(needs_layout_passes=False,
                                             use_tc_tiling_on_sc=False),
        out_type=[
            jax.ShapeDtypeStruct((2, nchunks, NPAD, CW), jnp.float32),
            jax.ShapeDtypeStruct((2, NPAD, CW), jnp.float32),
        ],
        mesh=mesh,
        scratch_types=[
            pltpu.VMEM((16,), jnp.float32),
            pltpu.VMEM((NBMAX * EB,), jnp.int32),
            pltpu.VMEM((NBMAX, EB), jnp.int32),
            pltpu.VMEM((heads, NBMAX * EB), jnp.float32),
            pltpu.VMEM((2, EB, CW), jnp.float32),
            pltpu.SemaphoreType.DMA((2,)),
            pltpu.SemaphoreType.DMA((2,)),
            pltpu.VMEM_SHARED((NPAD, CW), jnp.float32),
        ],
    )
    def sc_edge(xlc_hbm, tbl_hbm, gmax_hbm, src_hbm, dst_hbm, part_hbm, den_hbm,
                gmax_v, src_v, dst_v, w_v, mbufs,
                gsem, ssem, spacc):
        core = lax.axis_index("core")
        sub = lax.axis_index("subcore")
        nb = jnp.where(core == 0, NB0, NB1)
        boff = jnp.where(core == 0, sub * NB0, 16 * NB0 + sub * NB1)

        pltpu.sync_copy(gmax_hbm, gmax_v)
        pltpu.sync_copy(src_hbm.at[pl.ds(boff * EB, NBMAX * EB)], src_v)
        pltpu.sync_copy(dst_hbm.at[pl.ds(boff, NBMAX)], dst_v)

        z16 = jnp.zeros((16,), jnp.float32)
        gmvec = gmax_v[...]

        @pl.loop(0, EB)
        def _(r):
            for j in range(VPR):
                mbufs[0, r, pl.ds(j * 16, 16)] = z16
                mbufs[1, r, pl.ds(j * 16, 16)] = z16

        for k in range(ROWS_PT // EB):
            pltpu.sync_copy(mbufs.at[0], spacc.at[pl.ds(sub * ROWS_PT + k * EB, EB)])
        plsc.subcore_barrier()

        def _phase1(tbl_v):
            pltpu.sync_copy(tbl_hbm, tbl_v)

            @pl.loop(0, nb, step=2)
            def _(g):
                for k in range(2):
                    b = g + k

                    @pl.when(b >= 2)
                    def _():
                        pltpu.make_async_copy(mbufs.at[k], spacc.at[dst_v.at[b]],
                                              ssem.at[k]).wait()

                    for v in range(8):
                        sl = pl.ds(b * EB + v * 16, 16)
                        srcv = src_v[sl]
                        dstv = dst_v[b, pl.ds(v * 16, 16)]
                        rows = jnp.full((16,), v * 16, jnp.int32) + lax.iota(jnp.int32, 16)
                        for h in range(heads):
                            asrc = plsc.load_gather(tbl_v, [srcv + h * NPAD])
                            adst = plsc.load_gather(tbl_v, [dstv + (heads + h) * NPAD])
                            e = asrc + adst
                            e = jnp.where(e >= 0.0, e, 0.2 * e)
                            m = jnp.maximum(gmvec[h] + adst, 0.0)
                            w = jnp.exp(e - m)
                            w_v[h, sl] = w
                            plsc.store_scatter(
                                mbufs.at[k], [rows, jnp.full((16,), h, jnp.int32)], w)
                    pltpu.async_copy(mbufs.at[k], spacc.at[dst_v.at[b]], ssem.at[k],
                                     add=True)

        pl.run_scoped(_phase1, pltpu.VMEM((2 * heads * NPAD,), jnp.float32))

        for k in range(2):
            pltpu.make_async_copy(mbufs.at[k], spacc.at[dst_v.at[0]],
                                  ssem.at[k]).wait()
        plsc.subcore_barrier()
        pltpu.sync_copy(spacc.at[pl.ds(sub * ROWS_PT, ROWS_PT)],
                        den_hbm.at[core].at[pl.ds(sub * ROWS_PT, ROWS_PT)])

        def _phase2(gbufs):
            for c in range(nchunks):
                hh = c // cph

                def _mul(b, k, hh=hh):
                    @pl.loop(0, EB // 16)
                    def _(rb):
                        wv = w_v[hh, pl.ds(b * EB + rb * 16, 16)]
                        for l in range(16):
                            ws = wv[l]
                            for j in range(VPR):
                                slj = pl.ds(j * 16, 16)
                                mbufs[k, rb * 16 + l, slj] = (
                                    gbufs[k, rb * 16 + l, slj] * ws)

                @pl.loop(0, EB)
                def _(r):
                    for j in range(VPR):
                        mbufs[0, r, pl.ds(j * 16, 16)] = z16

                for k in range(ROWS_PT // EB):
                    pltpu.sync_copy(mbufs.at[0],
                                    spacc.at[pl.ds(sub * ROWS_PT + k * EB, EB)])
                plsc.subcore_barrier()

                for k in range(2):
                    pltpu.async_copy(xlc_hbm.at[c].at[src_v.at[pl.ds(k * EB, EB)]],
                                     gbufs.at[k], gsem.at[k])

                @pl.loop(0, nb - 2, step=2)
                def _(g):
                    for k in range(2):
                        b = g + k
                        pltpu.make_async_copy(
                            xlc_hbm.at[c].at[src_v.at[pl.ds(b * EB, EB)]],
                            gbufs.at[k], gsem.at[k]).wait()

                        @pl.when(b >= 2)
                        def _():
                            pltpu.make_async_copy(mbufs.at[k], spacc.at[dst_v.at[b]],
                                                  ssem.at[k]).wait()

                        _mul(b, k)
                        pltpu.async_copy(mbufs.at[k], spacc.at[dst_v.at[b]],
                                         ssem.at[k], add=True)
                        pltpu.async_copy(
                            xlc_hbm.at[c].at[src_v.at[pl.ds((b + 2) * EB, EB)]],
                            gbufs.at[k], gsem.at[k])

                for k in range(2):
                    b = nb - 2 + k
                    pltpu.make_async_copy(
                        xlc_hbm.at[c].at[src_v.at[pl.ds(b * EB, EB)]],
                        gbufs.at[k], gsem.at[k]).wait()
                    pltpu.make_async_copy(mbufs.at[k], spacc.at[dst_v.at[b]],
                                          ssem.at[k]).wait()
                    _mul(b, k)
                    pltpu.async_copy(mbufs.at[k], spacc.at[dst_v.at[b]],
                                     ssem.at[k], add=True)
                for k in range(2):
                    pltpu.make_async_copy(mbufs.at[k], spacc.at[dst_v.at[0]],
                                          ssem.at[k]).wait()

                plsc.subcore_barrier()
                pltpu.sync_copy(spacc.at[pl.ds(sub * ROWS_PT, ROWS_PT)],
                                part_hbm.at[core, c].at[pl.ds(sub * ROWS_PT, ROWS_PT)])

        pl.run_scoped(_phase2, pltpu.VMEM((2, EB, CW), jnp.float32))

    return sc_edge


_sc_edge_l1 = _make_sc_edge(HEADS, NC1)
_sc_edge_l2 = _make_sc_edge(1, NC2)

f32 = jnp.float32


def _run_mm1(x_pad, W1, A1):
    return pl.pallas_call(
        _mm1_body,
        grid=(GRID,),
        in_specs=[
            pl.BlockSpec((RB, D_IN), lambda i: (i, 0)),
            pl.BlockSpec((D_IN, 2 * HID), lambda i: (0, 0)),
            pl.BlockSpec((2 * HID, 128), lambda i: (0, 0)),
        ],
        out_specs=[
            pl.BlockSpec((NC1, RB, CW), lambda i: (0, i, 0)),
            pl.BlockSpec((RB, 128), lambda i: (i, 0)),
            pl.BlockSpec((8, 128), lambda i: (0, 0)),
        ],
        out_shape=[
            jax.ShapeDtypeStruct((NC1, NPAD, CW), f32),
            jax.ShapeDtypeStruct((NPAD, 128), f32),
            jax.ShapeDtypeStruct((8, 128), f32),
        ],
    )(x_pad, W1, A1)


def _run_mm2(parts1, dens1, b1m, W2, A2):
    return pl.pallas_call(
        _mm2_body,
        grid=(GRID,),
        in_specs=[
            pl.BlockSpec((2, NC1, RB, CW), lambda i: (0, 0, i, 0)),
            pl.BlockSpec((2, RB, CW), lambda i: (0, i, 0)),
            pl.BlockSpec((NC1, CW), lambda i: (0, 0)),
            pl.BlockSpec((2 * HID, OUT), lambda i: (0, 0)),
            pl.BlockSpec((OUT, 128), lambda i: (0, 0)),
        ],
        out_specs=[
            pl.BlockSpec((NC2, RB, CW), lambda i: (0, i, 0)),
            pl.BlockSpec((RB, 128), lambda i: (i, 0)),
            pl.BlockSpec((8, 128), lambda i: (0, 0)),
        ],
        out_shape=[
            jax.ShapeDtypeStruct((NC2, NPAD, CW), f32),
            jax.ShapeDtypeStruct((NPAD, 128), f32),
            jax.ShapeDtypeStruct((8, 128), f32),
        ],
    )(parts1, dens1, b1m, W2, A2)


def _run_fin(parts2, dens2, b2m):
    return pl.pallas_call(
        _fin_body,
        grid=(GRID,),
        in_specs=[
            pl.BlockSpec((2, NC2, RB, CW), lambda i: (0, 0, i, 0)),
            pl.BlockSpec((2, RB, CW), lambda i: (0, i, 0)),
            pl.BlockSpec((NC2, CW), lambda i: (0, 0)),
        ],
        out_specs=pl.BlockSpec((RB, OUT), lambda i: (i, 0)),
        out_shape=jax.ShapeDtypeStruct((NPAD, OUT), f32),
    )(parts2, dens2, b2m)


def kernel(x, edge_index, W1, att_src1, att_dst1, b1, W2, att_src2, att_dst2, b2):
    f32 = jnp.float32
    x_pad = jnp.pad(x, ((0, NPAD - N), (0, 0)))
    src = jnp.pad(edge_index[0], (0, EPAD_ARR - E))
    dst = jnp.pad(edge_index[1], (0, EPAD_ARR - E), constant_values=N)
    dst2d = dst.reshape(EPAD_ARR // EB, EB)

    as1 = att_src1.reshape(HEADS, HID)
    ad1 = att_dst1.reshape(HEADS, HID)
    z = jnp.zeros((HID,), f32)
    A1 = jnp.stack([
        jnp.concatenate([as1[0], z]), jnp.concatenate([z, as1[1]]),
        jnp.concatenate([ad1[0], z]), jnp.concatenate([z, ad1[1]]),
    ], axis=1)
    A1 = jnp.pad(A1, ((0, 0), (0, 124)))
    A2 = jnp.stack([att_src2.reshape(OUT), att_dst2.reshape(OUT)], axis=1)
    A2 = jnp.pad(A2, ((0, 0), (0, 126)))

    xlc1, ab1, gmax1 = _run_mm1(x_pad, W1, A1)

    tbl1 = jnp.concatenate([ab1[:, 0], ab1[:, 1], ab1[:, 2], ab1[:, 3]])
    gv1 = jnp.pad(gmax1[0, 0:HEADS], (0, 16 - HEADS))

    parts1, dens1 = _sc_edge_l1(xlc1, tbl1, gv1, src, dst2d)

    b1m = b1.reshape(NC1, CW)
    xlc2, ab2, gmax2 = _run_mm2(parts1, dens1, b1m, W2, A2)

    tbl2 = jnp.concatenate([ab2[:, 0], ab2[:, 1]])
    gv2 = jnp.pad(gmax2[0, 0:1], (0, 15))

    parts2, dens2 = _sc_edge_l2(xlc2, tbl2, gv2, src, dst2d)

    b2m = b2.reshape(NC2, CW)
    out = _run_fin(parts2, dens2, b2m)

    return out[:N]


# speedup vs baseline: 1.1306x; 1.1306x over previous
import functools

import jax
import jax.numpy as jnp
from jax import lax
from jax.experimental import pallas as pl
from jax.experimental.pallas import tpu as pltpu
from jax.experimental.pallas import tpu_sc as plsc

N = 10000
E = 160000
D_IN = 256
HID = 256
OUT = 256
HEADS = 2

NPAD = 10240
EB = 128
NB0 = 56
NB1 = 24
NBMAX = max(NB0, NB1)
EPAD = 16 * (NB0 + NB1) * EB
EPAD_ARR = EPAD + (NBMAX - min(NB0, NB1)) * EB
ROWS_PT = NPAD // 16
RB = 256
GRID = NPAD // RB
CW = 64
VPR = CW // 16
NC1 = 2 * HID // CW
NC2 = OUT // CW


def _mm1_body(x_ref, w_ref, a_ref, xlc_ref, ab_ref, gmax_ref):
    i = pl.program_id(0)
    xl = jnp.dot(x_ref[...], w_ref[...], preferred_element_type=jnp.float32)
    for c in range(NC1):
        xlc_ref[c] = xl[:, c * CW:(c + 1) * CW]
    ab = jnp.dot(xl, a_ref[...], preferred_element_type=jnp.float32)
    ab_ref[...] = ab
    bmax = jnp.broadcast_to(jnp.max(ab, axis=0, keepdims=True), (8, 128))

    @pl.when(i == 0)
    def _():
        gmax_ref[...] = bmax

    @pl.when(i != 0)
    def _():
        gmax_ref[...] = jnp.maximum(gmax_ref[...], bmax)


def _mm2_body(p_ref, d_ref, b_ref, w_ref, a_ref, xlc_ref, ab_ref, gmax_ref):
    i = pl.program_id(0)
    d = d_ref[0] + d_ref[1]
    rden = 1.0 / (d + 1e-30)
    acc = jnp.zeros((RB, OUT), jnp.float32)
    for c in range(NC1):
        hh = c // (NC1 // 2)
        hc = (p_ref[0, c] + p_ref[1, c]) * rden[:, hh:hh + 1] + b_ref[c][None, :]
        hc = jnp.where(hc > 0.0, hc, jnp.exp(jnp.minimum(hc, 0.0)) - 1.0)
        acc = acc + jnp.dot(hc, w_ref[pl.ds(c * CW, CW), :],
                            preferred_element_type=jnp.float32)
    for c in range(NC2):
        xlc_ref[c] = acc[:, c * CW:(c + 1) * CW]
    ab = jnp.dot(acc, a_ref[...], preferred_element_type=jnp.float32)
    ab_ref[...] = ab
    bmax = jnp.broadcast_to(jnp.max(ab, axis=0, keepdims=True), (8, 128))

    @pl.when(i == 0)
    def _():
        gmax_ref[...] = bmax

    @pl.when(i != 0)
    def _():
        gmax_ref[...] = jnp.maximum(gmax_ref[...], bmax)


def _fin_body(p_ref, d_ref, b_ref, o_ref):
    d = d_ref[0] + d_ref[1]
    rden = 1.0 / (d + 1e-30)
    for c in range(NC2):
        pc = p_ref[0, c] + p_ref[1, c]
        o_ref[:, c * CW:(c + 1) * CW] = pc * rden[:, 0:1] + b_ref[c][None, :]


def _make_sc_edge(heads, nchunks):
    cph = nchunks // heads
    mesh = plsc.VectorSubcoreMesh(core_axis_name="core", subcore_axis_name="subcore",
                                  num_cores=2, num_subcores=16)

    @functools.partial(
        pl.kernel,
        compiler_params=pltpu.CompilerParams(needs_layout_passes=False,
                                             use_tc_tiling_on_sc=False),
        out_type=[
            jax.ShapeDtypeStruct((2, nchunks, NPAD, CW), jnp.float32),
            jax.ShapeDtypeStruct((2, NPAD, CW), jnp.float32),
        ],
        mesh=mesh,
        scratch_types=[
            pltpu.VMEM((16,), jnp.float32),
            pltpu.VMEM((NBMAX * EB,), jnp.int32),
            pltpu.VMEM((NBMAX, EB), jnp.int32),
            pltpu.VMEM((heads, NBMAX * EB), jnp.float32),
            pltpu.VMEM((2, EB, CW), jnp.float32),
            pltpu.SemaphoreType.DMA((2,)),
            pltpu.SemaphoreType.DMA((2,)),
            pltpu.VMEM_SHARED((NPAD, CW), jnp.float32),
        ],
    )
    def sc_edge(xlc_hbm, tbl_hbm, gmax_hbm, src_hbm, dst_hbm, part_hbm, den_hbm,
                gmax_v, src_v, dst_v, w_v, mbufs,
                gsem, ssem, spacc):
        core = lax.axis_index("core")
        sub = lax.axis_index("subcore")
        nb = jnp.where(core == 0, NB0, NB1)
        boff = jnp.where(core == 0, sub * NB0, 16 * NB0 + sub * NB1)

        pltpu.sync_copy(gmax_hbm, gmax_v)
        pltpu.sync_copy(src_hbm.at[pl.ds(boff * EB, NBMAX * EB)], src_v)
        pltpu.sync_copy(dst_hbm.at[pl.ds(boff, NBMAX)], dst_v)

        z16 = jnp.zeros((16,), jnp.float32)
        gmvec = gmax_v[...]

        @pl.loop(0, EB)
        def _(r):
            for j in range(VPR):
                mbufs[0, r, pl.ds(j * 16, 16)] = z16
                mbufs[1, r, pl.ds(j * 16, 16)] = z16

        for k in range(ROWS_PT // EB):
            pltpu.sync_copy(mbufs.at[0], spacc.at[pl.ds(sub * ROWS_PT + k * EB, EB)])
        plsc.subcore_barrier()

        def _phase1(tbl_v):
            pltpu.sync_copy(tbl_hbm, tbl_v)

            @pl.loop(0, nb, step=2)
            def _(g):
                for k in range(2):
                    b = g + k

                    @pl.when(b >= 2)
                    def _():
                        pltpu.make_async_copy(mbufs.at[k], spacc.at[dst_v.at[b]],
                                              ssem.at[k]).wait()

                    for v in range(8):
                        sl = pl.ds(b * EB + v * 16, 16)
                        srcv = src_v[sl]
                        dstv = dst_v[b, pl.ds(v * 16, 16)]
                        rows = jnp.full((16,), v * 16, jnp.int32) + lax.iota(jnp.int32, 16)
                        for h in range(heads):
                            asrc = plsc.load_gather(tbl_v, [srcv + h * NPAD])
                            adst = plsc.load_gather(tbl_v, [dstv + (heads + h) * NPAD])
                            e = asrc + adst
                            e = jnp.where(e >= 0.0, e, 0.2 * e)
                            m = jnp.maximum(gmvec[h] + adst, 0.0)
                            w = jnp.exp(e - m)
                            w_v[h, sl] = w
                            plsc.store_scatter(
                                mbufs.at[k], [rows, jnp.full((16,), h, jnp.int32)], w)
                    pltpu.async_copy(mbufs.at[k], spacc.at[dst_v.at[b]], ssem.at[k],
                                     add=True)

        pl.run_scoped(_phase1, pltpu.VMEM((2 * heads * NPAD,), jnp.float32))

        for k in range(2):
            pltpu.make_async_copy(mbufs.at[k], spacc.at[dst_v.at[0]],
                                  ssem.at[k]).wait()
        plsc.subcore_barrier()
        pltpu.sync_copy(spacc.at[pl.ds(sub * ROWS_PT, ROWS_PT)],
                        den_hbm.at[core].at[pl.ds(sub * ROWS_PT, ROWS_PT)])

        def _phase2(gbufs):
            for c in range(nchunks):
                hh = c // cph

                def _mul(b, k, hh=hh):
                    @pl.loop(0, EB // 16)
                    def _(rb):
                        wv = w_v[hh, pl.ds(b * EB + rb * 16, 16)]
                        for l in range(16):
                            ws = wv[l]
                            for j in range(VPR):
                                slj = pl.ds(j * 16, 16)
                                mbufs[k, rb * 16 + l, slj] = (
                                    gbufs[k, rb * 16 + l, slj] * ws)

                @pl.loop(0, EB)
                def _(r):
                    for j in range(VPR):
                        mbufs[0, r, pl.ds(j * 16, 16)] = z16

                for k in range(ROWS_PT // EB):
                    pltpu.sync_copy(mbufs.at[0],
                                    spacc.at[pl.ds(sub * ROWS_PT + k * EB, EB)])
                plsc.subcore_barrier()

                for k in range(2):
                    pltpu.async_copy(xlc_hbm.at[c].at[src_v.at[pl.ds(k * EB, EB)]],
                                     gbufs.at[k], gsem.at[k])

                @pl.loop(0, nb - 2, step=2)
                def _(g):
                    for k in range(2):
                        b = g + k
                        pltpu.make_async_copy(
                            xlc_hbm.at[c].at[src_v.at[pl.ds(b * EB, EB)]],
                            gbufs.at[k], gsem.at[k]).wait()

                        @pl.when(b >= 2)
                        def _():
                            pltpu.make_async_copy(mbufs.at[k], spacc.at[dst_v.at[b]],
                                                  ssem.at[k]).wait()

                        _mul(b, k)
                        pltpu.async_copy(mbufs.at[k], spacc.at[dst_v.at[b]],
                                         ssem.at[k], add=True)
                        pltpu.async_copy(
                            xlc_hbm.at[c].at[src_v.at[pl.ds((b + 2) * EB, EB)]],
                            gbufs.at[k], gsem.at[k])

                for k in range(2):
                    b = nb - 2 + k
                    pltpu.make_async_copy(
                        xlc_hbm.at[c].at[src_v.at[pl.ds(b * EB, EB)]],
                        gbufs.at[k], gsem.at[k]).wait()
                    pltpu.make_async_copy(mbufs.at[k], spacc.at[dst_v.at[b]],
                                          ssem.at[k]).wait()
                    _mul(b, k)
                    pltpu.async_copy(mbufs.at[k], spacc.at[dst_v.at[b]],
                                     ssem.at[k], add=True)
                for k in range(2):
                    pltpu.make_async_copy(mbufs.at[k], spacc.at[dst_v.at[0]],
                                          ssem.at[k]).wait()

                plsc.subcore_barrier()
                pltpu.sync_copy(spacc.at[pl.ds(sub * ROWS_PT, ROWS_PT)],
                                part_hbm.at[core, c].at[pl.ds(sub * ROWS_PT, ROWS_PT)])

        pl.run_scoped(_phase2, pltpu.VMEM((2, EB, CW), jnp.float32))

    return sc_edge


_sc_edge_l1 = _make_sc_edge(HEADS, NC1)
_sc_edge_l2 = _make_sc_edge(1, NC2)

f32 = jnp.float32


def _run_mm1(x_pad, W1, A1):
    return pl.pallas_call(
        _mm1_body,
        grid=(GRID,),
        in_specs=[
            pl.BlockSpec((RB, D_IN), lambda i: (i, 0)),
            pl.BlockSpec((D_IN, 2 * HID), lambda i: (0, 0)),
            pl.BlockSpec((2 * HID, 128), lambda i: (0, 0)),
        ],
        out_specs=[
            pl.BlockSpec((NC1, RB, CW), lambda i: (0, i, 0)),
            pl.BlockSpec((RB, 128), lambda i: (i, 0)),
            pl.BlockSpec((8, 128), lambda i: (0, 0)),
        ],
        out_shape=[
            jax.ShapeDtypeStruct((NC1, NPAD, CW), f32),
            jax.ShapeDtypeStruct((NPAD, 128), f32),
            jax.ShapeDtypeStruct((8, 128), f32),
        ],
    )(x_pad, W1, A1)


def _run_mm2(parts1, dens1, b1m, W2, A2):
    return pl.pallas_call(
        _mm2_body,
        grid=(GRID,),
        in_specs=[
            pl.BlockSpec((2, NC1, RB, CW), lambda i: (0, 0, i, 0)),
            pl.BlockSpec((2, RB, CW), lambda i: (0, i, 0)),
            pl.BlockSpec((NC1, CW), lambda i: (0, 0)),
            pl.BlockSpec((2 * HID, OUT), lambda i: (0, 0)),
            pl.BlockSpec((OUT, 128), lambda i: (0, 0)),
        ],
        out_specs=[
            pl.BlockSpec((NC2, RB, CW), lambda i: (0, i, 0)),
            pl.BlockSpec((RB, 128), lambda i: (i, 0)),
            pl.BlockSpec((8, 128), lambda i: (0, 0)),
        ],
        out_shape=[
            jax.ShapeDtypeStruct((NC2, NPAD, CW), f32),
            jax.ShapeDtypeStruct((NPAD, 128), f32),
            jax.ShapeDtypeStruct((8, 128), f32),
        ],
    )(parts1, dens1, b1m, W2, A2)


def _run_fin(parts2, dens2, b2m):
    return pl.pallas_call(
        _fin_body,
        grid=(GRID,),
        in_specs=[
            pl.BlockSpec((2, NC2, RB, CW), lambda i: (0, 0, i, 0)),
            pl.BlockSpec((2, RB, CW), lambda i: (0, i, 0)),
            pl.BlockSpec((NC2, CW), lambda i: (0, 0)),
        ],
        out_specs=pl.BlockSpec((RB, OUT), lambda i: (i, 0)),
        out_shape=jax.ShapeDtypeStruct((NPAD, OUT), f32),
    )(parts2, dens2, b2m)


def kernel(x, edge_index, W1, att_src1, att_dst1, b1, W2, att_src2, att_dst2, b2):
    f32 = jnp.float32
    x_pad = jnp.pad(x, ((0, NPAD - N), (0, 0)))
    src = jnp.pad(edge_index[0], (0, EPAD_ARR - E))
    dst = jnp.pad(edge_index[1], (0, EPAD_ARR - E), constant_values=N)
    dst2d = dst.reshape(EPAD_ARR // EB, EB)

    as1 = att_src1.reshape(HEADS, HID)
    ad1 = att_dst1.reshape(HEADS, HID)
    z = jnp.zeros((HID,), f32)
    A1 = jnp.stack([
        jnp.concatenate([as1[0], z]), jnp.concatenate([z, as1[1]]),
        jnp.concatenate([ad1[0], z]), jnp.concatenate([z, ad1[1]]),
    ], axis=1)
    A1 = jnp.pad(A1, ((0, 0), (0, 124)))
    A2 = jnp.stack([att_src2.reshape(OUT), att_dst2.reshape(OUT)], axis=1)
    A2 = jnp.pad(A2, ((0, 0), (0, 126)))

    xlc1, ab1, gmax1 = _run_mm1(x_pad, W1, A1)

    tbl1 = jnp.concatenate([ab1[:, 0], ab1[:, 1], ab1[:, 2], ab1[:, 3]])
    gv1 = jnp.pad(gmax1[0, 0:HEADS], (0, 16 - HEADS))

    parts1, dens1 = _sc_edge_l1(xlc1, tbl1, gv1, src, dst2d)

    b1m = b1.reshape(NC1, CW)
    xlc2, ab2, gmax2 = _run_mm2(parts1, dens1, b1m, W2, A2)

    tbl2 = jnp.concatenate([ab2[:, 0], ab2[:, 1]])
    gv2 = jnp.pad(gmax2[0, 0:1], (0, 15))

    parts2, dens2 = _sc_edge_l2(xlc2, tbl2, gv2, src, dst2d)

    b2m = b2.reshape(NC2, CW)
    out = _run_fin(parts2, dens2, b2m)

    return out[:N]


# speedup vs baseline: 1.1567x; 1.0232x over previous
import functools

import jax
import jax.numpy as jnp
from jax import lax
from jax.experimental import pallas as pl
from jax.experimental.pallas import tpu as pltpu
from jax.experimental.pallas import tpu_sc as plsc

N = 10000
E = 160000
D_IN = 256
HID = 256
OUT = 256
HEADS = 2

NPAD = 10240
EB = 128
NB0 = 64
NB1 = 16
NBMAX = max(NB0, NB1)
EPAD = 16 * (NB0 + NB1) * EB
EPAD_ARR = EPAD + (NBMAX - min(NB0, NB1)) * EB
ROWS_PT = NPAD // 16
RB = 256
GRID = NPAD // RB
CW = 64
VPR = CW // 16
NC1 = 2 * HID // CW
NC2 = OUT // CW


def _mm1_body(x_ref, w_ref, a_ref, xlc_ref, ab_ref, gmax_ref):
    i = pl.program_id(0)
    xl = jnp.dot(x_ref[...], w_ref[...], preferred_element_type=jnp.float32)
    for c in range(NC1):
        xlc_ref[c] = xl[:, c * CW:(c + 1) * CW]
    ab = jnp.dot(xl, a_ref[...], preferred_element_type=jnp.float32)
    ab_ref[...] = ab
    bmax = jnp.broadcast_to(jnp.max(ab, axis=0, keepdims=True), (8, 128))

    @pl.when(i == 0)
    def _():
        gmax_ref[...] = bmax

    @pl.when(i != 0)
    def _():
        gmax_ref[...] = jnp.maximum(gmax_ref[...], bmax)


def _mm2_body(p_ref, d_ref, b_ref, w_ref, a_ref, xlc_ref, ab_ref, gmax_ref):
    i = pl.program_id(0)
    d = d_ref[0] + d_ref[1]
    rden = 1.0 / (d + 1e-30)
    acc = jnp.zeros((RB, OUT), jnp.float32)
    for c in range(NC1):
        hh = c // (NC1 // 2)
        hc = (p_ref[0, c] + p_ref[1, c]) * rden[:, hh:hh + 1] + b_ref[c][None, :]
        hc = jnp.where(hc > 0.0, hc, jnp.exp(jnp.minimum(hc, 0.0)) - 1.0)
        acc = acc + jnp.dot(hc, w_ref[pl.ds(c * CW, CW), :],
                            preferred_element_type=jnp.float32)
    for c in range(NC2):
        xlc_ref[c] = acc[:, c * CW:(c + 1) * CW]
    ab = jnp.dot(acc, a_ref[...], preferred_element_type=jnp.float32)
    ab_ref[...] = ab
    bmax = jnp.broadcast_to(jnp.max(ab, axis=0, keepdims=True), (8, 128))

    @pl.when(i == 0)
    def _():
        gmax_ref[...] = bmax

    @pl.when(i != 0)
    def _():
        gmax_ref[...] = jnp.maximum(gmax_ref[...], bmax)


def _fin_body(p_ref, d_ref, b_ref, o_ref):
    d = d_ref[0] + d_ref[1]
    rden = 1.0 / (d + 1e-30)
    for c in range(NC2):
        pc = p_ref[0, c] + p_ref[1, c]
        o_ref[:, c * CW:(c + 1) * CW] = pc * rden[:, 0:1] + b_ref[c][None, :]


def _make_sc_edge(heads, nchunks):
    cph = nchunks // heads
    tslab = (2 * heads * NPAD + EB * CW - 1) // (EB * CW)
    mesh = plsc.VectorSubcoreMesh(core_axis_name="core", subcore_axis_name="subcore",
                                  num_cores=2, num_subcores=16)

    @functools.partial(
        pl.kernel,
        compiler_params=pltpu.CompilerParams(needs_layout_passes=False,
                                             use_tc_tiling_on_sc=False),
        out_type=[
            jax.ShapeDtypeStruct((2, nchunks, NPAD, CW), jnp.float32),
            jax.ShapeDtypeStruct((2, NPAD, CW), jnp.float32),
        ],
        mesh=mesh,
        scratch_types=[
            pltpu.VMEM((NBMAX * EB,), jnp.int32),
            pltpu.VMEM((NBMAX, EB), jnp.int32),
            pltpu.VMEM((heads, NBMAX * EB), jnp.float32),
            pltpu.VMEM((2, EB, CW), jnp.float32),
            pltpu.SemaphoreType.DMA((2,)),
            pltpu.SemaphoreType.DMA((2,)),
            pltpu.VMEM_SHARED((NPAD, CW), jnp.float32),
        ],
    )
    def sc_edge(xlc_hbm, tbl_hbm, gmax_hbm, src_hbm, dst_hbm, part_hbm, den_hbm,
                src_v, dst_v, w_v, mbufs,
                gsem, ssem, spacc):
        core = lax.axis_index("core")
        sub = lax.axis_index("subcore")
        nb = jnp.where(core == 0, NB0, NB1)
        boff = jnp.where(core == 0, sub * NB0, 16 * NB0 + sub * NB1)

        pltpu.sync_copy(src_hbm.at[pl.ds(boff * EB, NBMAX * EB)], src_v)
        pltpu.sync_copy(dst_hbm.at[pl.ds(boff, NBMAX)], dst_v)

        z16 = jnp.zeros((16,), jnp.float32)
        pltpu.sync_copy(gmax_hbm, mbufs.at[0, 0, pl.ds(0, 16)])
        gv = mbufs[0, 0, pl.ds(0, 16)]
        gmvec = [gv[h] for h in range(heads)]

        @pl.loop(0, EB)
        def _(r):
            for j in range(VPR):
                mbufs[0, r, pl.ds(j * 16, 16)] = z16
                mbufs[1, r, pl.ds(j * 16, 16)] = z16

        for k in range(ROWS_PT // EB):
            pltpu.sync_copy(mbufs.at[0], spacc.at[pl.ds(sub * ROWS_PT + k * EB, EB)])
        plsc.subcore_barrier()

        def _phases(buf):
            pltpu.sync_copy(tbl_hbm, buf)

            def _tbl_gather(idx):
                return plsc.load_gather(
                    buf, [idx >> 13, (idx >> 6) & 127, idx & 63])

            @pl.loop(0, nb, step=2)
            def _(g):
                for k in range(2):
                    b = g + k

                    @pl.when(b >= 2)
                    def _():
                        pltpu.make_async_copy(mbufs.at[k], spacc.at[dst_v.at[b]],
                                              ssem.at[k]).wait()

                    for v in range(8):
                        sl = pl.ds(b * EB + v * 16, 16)
                        srcv = src_v[sl]
                        dstv = dst_v[b, pl.ds(v * 16, 16)]
                        rows = jnp.full((16,), v * 16, jnp.int32) + lax.iota(jnp.int32, 16)
                        for h in range(heads):
                            asrc = _tbl_gather(srcv + h * NPAD)
                            adst = _tbl_gather(dstv + (heads + h) * NPAD)
                            e = asrc + adst
                            e = jnp.where(e >= 0.0, e, 0.2 * e)
                            m = jnp.maximum(gmvec[h] + adst, 0.0)
                            w = jnp.exp(e - m)
                            w_v[h, sl] = w
                            plsc.store_scatter(
                                mbufs.at[k], [rows, jnp.full((16,), h, jnp.int32)], w)
                    pltpu.async_copy(mbufs.at[k], spacc.at[dst_v.at[b]], ssem.at[k],
                                     add=True)

            for k in range(2):
                pltpu.make_async_copy(mbufs.at[k], spacc.at[dst_v.at[0]],
                                      ssem.at[k]).wait()
            plsc.subcore_barrier()
            pltpu.sync_copy(spacc.at[pl.ds(sub * ROWS_PT, ROWS_PT)],
                            den_hbm.at[core].at[pl.ds(sub * ROWS_PT, ROWS_PT)])

            for c in range(nchunks):
                hh = c // cph

                def _mul(b, k, hh=hh):
                    @pl.loop(0, EB // 16)
                    def _(rb):
                        wv = w_v[hh, pl.ds(b * EB + rb * 16, 16)]
                        for l in range(16):
                            ws = wv[l]
                            for j in range(VPR):
                                slj = pl.ds(j * 16, 16)
                                mbufs[k, rb * 16 + l, slj] = (
                                    buf[k, rb * 16 + l, slj] * ws)

                @pl.loop(0, EB)
                def _(r):
                    for j in range(VPR):
                        mbufs[0, r, pl.ds(j * 16, 16)] = z16

                for k in range(ROWS_PT // EB):
                    pltpu.sync_copy(mbufs.at[0],
                                    spacc.at[pl.ds(sub * ROWS_PT + k * EB, EB)])
                plsc.subcore_barrier()

                for k in range(2):
                    pltpu.async_copy(xlc_hbm.at[c].at[src_v.at[pl.ds(k * EB, EB)]],
                                     buf.at[k], gsem.at[k])

                @pl.loop(0, nb - 2, step=2)
                def _(g):
                    for k in range(2):
                        b = g + k
                        pltpu.make_async_copy(
                            xlc_hbm.at[c].at[src_v.at[pl.ds(b * EB, EB)]],
                            buf.at[k], gsem.at[k]).wait()

                        @pl.when(b >= 2)
                        def _():
                            pltpu.make_async_copy(mbufs.at[k], spacc.at[dst_v.at[b]],
                                                  ssem.at[k]).wait()

                        _mul(b, k)
                        pltpu.async_copy(mbufs.at[k], spacc.at[dst_v.at[b]],
                                         ssem.at[k], add=True)
                        pltpu.async_copy(
                            xlc_hbm.at[c].at[src_v.at[pl.ds((b + 2) * EB, EB)]],
                            buf.at[k], gsem.at[k])

                for k in range(2):
                    b = nb - 2 + k
                    pltpu.make_async_copy(
                        xlc_hbm.at[c].at[src_v.at[pl.ds(b * EB, EB)]],
                        buf.at[k], gsem.at[k]).wait()
                    pltpu.make_async_copy(mbufs.at[k], spacc.at[dst_v.at[b]],
                                          ssem.at[k]).wait()
                    _mul(b, k)
                    pltpu.async_copy(mbufs.at[k], spacc.at[dst_v.at[b]],
                                     ssem.at[k], add=True)
                for k in range(2):
                    pltpu.make_async_copy(mbufs.at[k], spacc.at[dst_v.at[0]],
                                          ssem.at[k]).wait()

                plsc.subcore_barrier()
                pltpu.sync_copy(spacc.at[pl.ds(sub * ROWS_PT, ROWS_PT)],
                                part_hbm.at[core, c].at[pl.ds(sub * ROWS_PT, ROWS_PT)])

        pl.run_scoped(_phases, pltpu.VMEM((tslab, EB, CW), jnp.float32))

    return sc_edge


_sc_edge_l1 = _make_sc_edge(HEADS, NC1)
_sc_edge_l2 = _make_sc_edge(1, NC2)

f32 = jnp.float32


def _run_mm1(x_pad, W1, A1):
    return pl.pallas_call(
        _mm1_body,
        grid=(GRID,),
        in_specs=[
            pl.BlockSpec((RB, D_IN), lambda i: (i, 0)),
            pl.BlockSpec((D_IN, 2 * HID), lambda i: (0, 0)),
            pl.BlockSpec((2 * HID, 128), lambda i: (0, 0)),
        ],
        out_specs=[
            pl.BlockSpec((NC1, RB, CW), lambda i: (0, i, 0)),
            pl.BlockSpec((RB, 128), lambda i: (i, 0)),
            pl.BlockSpec((8, 128), lambda i: (0, 0)),
        ],
        out_shape=[
            jax.ShapeDtypeStruct((NC1, NPAD, CW), f32),
            jax.ShapeDtypeStruct((NPAD, 128), f32),
            jax.ShapeDtypeStruct((8, 128), f32),
        ],
    )(x_pad, W1, A1)


def _run_mm2(parts1, dens1, b1m, W2, A2):
    return pl.pallas_call(
        _mm2_body,
        grid=(GRID,),
        in_specs=[
            pl.BlockSpec((2, NC1, RB, CW), lambda i: (0, 0, i, 0)),
            pl.BlockSpec((2, RB, CW), lambda i: (0, i, 0)),
            pl.BlockSpec((NC1, CW), lambda i: (0, 0)),
            pl.BlockSpec((2 * HID, OUT), lambda i: (0, 0)),
            pl.BlockSpec((OUT, 128), lambda i: (0, 0)),
        ],
        out_specs=[
            pl.BlockSpec((NC2, RB, CW), lambda i: (0, i, 0)),
            pl.BlockSpec((RB, 128), lambda i: (i, 0)),
            pl.BlockSpec((8, 128), lambda i: (0, 0)),
        ],
        out_shape=[
            jax.ShapeDtypeStruct((NC2, NPAD, CW), f32),
            jax.ShapeDtypeStruct((NPAD, 128), f32),
            jax.ShapeDtypeStruct((8, 128), f32),
        ],
    )(parts1, dens1, b1m, W2, A2)


def _run_fin(parts2, dens2, b2m):
    return pl.pallas_call(
        _fin_body,
        grid=(GRID,),
        in_specs=[
            pl.BlockSpec((2, NC2, RB, CW), lambda i: (0, 0, i, 0)),
            pl.BlockSpec((2, RB, CW), lambda i: (0, i, 0)),
            pl.BlockSpec((NC2, CW), lambda i: (0, 0)),
        ],
        out_specs=pl.BlockSpec((RB, OUT), lambda i: (i, 0)),
        out_shape=jax.ShapeDtypeStruct((NPAD, OUT), f32),
    )(parts2, dens2, b2m)


def kernel(x, edge_index, W1, att_src1, att_dst1, b1, W2, att_src2, att_dst2, b2):
    f32 = jnp.float32
    x_pad = jnp.pad(x, ((0, NPAD - N), (0, 0)))
    src = jnp.pad(edge_index[0], (0, EPAD_ARR - E))
    dst = jnp.pad(edge_index[1], (0, EPAD_ARR - E), constant_values=N)
    dst2d = dst.reshape(EPAD_ARR // EB, EB)

    as1 = att_src1.reshape(HEADS, HID)
    ad1 = att_dst1.reshape(HEADS, HID)
    z = jnp.zeros((HID,), f32)
    A1 = jnp.stack([
        jnp.concatenate([as1[0], z]), jnp.concatenate([z, as1[1]]),
        jnp.concatenate([ad1[0], z]), jnp.concatenate([z, ad1[1]]),
    ], axis=1)
    A1 = jnp.pad(A1, ((0, 0), (0, 124)))
    A2 = jnp.stack([att_src2.reshape(OUT), att_dst2.reshape(OUT)], axis=1)
    A2 = jnp.pad(A2, ((0, 0), (0, 126)))

    xlc1, ab1, gmax1 = _run_mm1(x_pad, W1, A1)

    tbl1 = jnp.concatenate([ab1[:, 0], ab1[:, 1], ab1[:, 2], ab1[:, 3]])
    tbl1 = tbl1.reshape(-1, EB, CW)
    gv1 = jnp.pad(gmax1[0, 0:HEADS], (0, 16 - HEADS))

    parts1, dens1 = _sc_edge_l1(xlc1, tbl1, gv1, src, dst2d)

    b1m = b1.reshape(NC1, CW)
    xlc2, ab2, gmax2 = _run_mm2(parts1, dens1, b1m, W2, A2)

    tbl2 = jnp.pad(jnp.concatenate([ab2[:, 0], ab2[:, 1]]),
                   (0, 3 * EB * CW - 2 * NPAD)).reshape(-1, EB, CW)
    gv2 = jnp.pad(gmax2[0, 0:1], (0, 15))

    parts2, dens2 = _sc_edge_l2(xlc2, tbl2, gv2, src, dst2d)

    b2m = b2.reshape(NC2, CW)
    out = _run_fin(parts2, dens2, b2m)

    return out[:N]


# speedup vs baseline: 1.1805x; 1.0205x over previous
import functools

import jax
import jax.numpy as jnp
from jax import lax
from jax.experimental import pallas as pl
from jax.experimental.pallas import tpu as pltpu
from jax.experimental.pallas import tpu_sc as plsc

N = 10000
E = 160000
D_IN = 256
HID = 256
OUT = 256
HEADS = 2

NPAD = 10240
EB = 128
NB0 = 64
NB1 = 16
NBMAX = max(NB0, NB1)
EPAD = 16 * (NB0 + NB1) * EB
EPAD_ARR = EPAD + (NBMAX - min(NB0, NB1)) * EB
ROWS_PT = NPAD // 16
RB = 256
GRID = NPAD // RB
CW = 64
VPR = CW // 16
NC1 = 2 * HID // CW
NC2 = OUT // CW


def _mm1_body(x_ref, w_ref, a_ref, xlc_ref, ab_ref, gmax_ref):
    i = pl.program_id(0)
    xl = jnp.dot(x_ref[...], w_ref[...], preferred_element_type=jnp.float32)
    for c in range(NC1):
        xlc_ref[c] = xl[:, c * CW:(c + 1) * CW]
    ab = jnp.dot(xl, a_ref[...], preferred_element_type=jnp.float32)
    ab_ref[...] = ab
    bmax = jnp.broadcast_to(jnp.max(ab, axis=0, keepdims=True), (8, 128))

    @pl.when(i == 0)
    def _():
        gmax_ref[...] = bmax

    @pl.when(i != 0)
    def _():
        gmax_ref[...] = jnp.maximum(gmax_ref[...], bmax)


def _mm2_body(p_ref, d_ref, b_ref, w_ref, a_ref, xlc_ref, ab_ref, gmax_ref):
    i = pl.program_id(0)
    d = d_ref[0] + d_ref[1]
    rden = 1.0 / (d + 1e-30)
    acc = jnp.zeros((RB, OUT), jnp.float32)
    for c in range(NC1):
        hh = c // (NC1 // 2)
        hc = (p_ref[0, c] + p_ref[1, c]) * rden[:, hh:hh + 1] + b_ref[c][None, :]
        hc = jnp.where(hc > 0.0, hc, jnp.exp(jnp.minimum(hc, 0.0)) - 1.0)
        acc = acc + jnp.dot(hc, w_ref[pl.ds(c * CW, CW), :],
                            preferred_element_type=jnp.float32)
    for c in range(NC2):
        xlc_ref[c] = acc[:, c * CW:(c + 1) * CW]
    ab = jnp.dot(acc, a_ref[...], preferred_element_type=jnp.float32)
    ab_ref[...] = ab
    bmax = jnp.broadcast_to(jnp.max(ab, axis=0, keepdims=True), (8, 128))

    @pl.when(i == 0)
    def _():
        gmax_ref[...] = bmax

    @pl.when(i != 0)
    def _():
        gmax_ref[...] = jnp.maximum(gmax_ref[...], bmax)


def _fin_body(p_ref, d_ref, b_ref, o_ref):
    d = d_ref[0] + d_ref[1]
    rden = 1.0 / (d + 1e-30)
    for c in range(NC2):
        pc = p_ref[0, c] + p_ref[1, c]
        o_ref[:, c * CW:(c + 1) * CW] = pc * rden[:, 0:1] + b_ref[c][None, :]


def _make_sc_edge(heads, nchunks):
    cph = nchunks // heads
    tslab = (2 * heads * NPAD + EB * CW - 1) // (EB * CW)
    mesh = plsc.VectorSubcoreMesh(core_axis_name="core", subcore_axis_name="subcore",
                                  num_cores=2, num_subcores=16)

    @functools.partial(
        pl.kernel,
        compiler_params=pltpu.CompilerParams(needs_layout_passes=False,
                                             use_tc_tiling_on_sc=False),
        out_type=[
            jax.ShapeDtypeStruct((2, nchunks, NPAD, CW), jnp.float32),
            jax.ShapeDtypeStruct((2, NPAD, CW), jnp.float32),
        ],
        mesh=mesh,
        scratch_types=[
            pltpu.VMEM((NBMAX * EB,), jnp.int32),
            pltpu.VMEM((NBMAX, EB), jnp.int32),
            pltpu.VMEM((heads, NBMAX * EB), jnp.float32),
            pltpu.VMEM((2, EB, CW), jnp.float32),
            pltpu.SemaphoreType.DMA((2,)),
            pltpu.SemaphoreType.DMA((2,)),
            pltpu.SemaphoreType.DMA,
            pltpu.VMEM_SHARED((NPAD, CW), jnp.float32),
        ],
    )
    def sc_edge(xlc_hbm, tbl_hbm, gmax_hbm, src_hbm, dst_hbm, part_hbm, den_hbm,
                src_v, dst_v, w_v, mbufs,
                gsem, ssem, zsem, spacc):
        core = lax.axis_index("core")
        sub = lax.axis_index("subcore")
        nb = jnp.where(core == 0, NB0, NB1)
        boff = jnp.where(core == 0, sub * NB0, 16 * NB0 + sub * NB1)

        pltpu.sync_copy(src_hbm.at[pl.ds(boff * EB, NBMAX * EB)], src_v)
        pltpu.sync_copy(dst_hbm.at[pl.ds(boff, NBMAX)], dst_v)

        z16 = jnp.zeros((16,), jnp.float32)
        pltpu.sync_copy(gmax_hbm, mbufs.at[0, 0, pl.ds(0, 16)])
        gv = mbufs[0, 0, pl.ds(0, 16)]
        gmvec = [gv[h] for h in range(heads)]

        @pl.loop(0, EB)
        def _(r):
            for j in range(VPR):
                mbufs[0, r, pl.ds(j * 16, 16)] = z16
                mbufs[1, r, pl.ds(j * 16, 16)] = z16

        for k in range(ROWS_PT // EB):
            pltpu.async_copy(mbufs.at[0],
                             spacc.at[pl.ds(sub * ROWS_PT + k * EB, EB)], zsem)
        for k in range(ROWS_PT // EB):
            pltpu.make_async_copy(
                mbufs.at[0], spacc.at[pl.ds(sub * ROWS_PT + k * EB, EB)],
                zsem).wait()
        plsc.subcore_barrier()

        def _phases(buf):
            pltpu.sync_copy(tbl_hbm, buf)

            def _tbl_gather(idx):
                return plsc.load_gather(
                    buf, [idx >> 13, (idx >> 6) & 127, idx & 63])

            @pl.loop(0, nb, step=2)
            def _(g):
                for k in range(2):
                    b = g + k

                    @pl.when(b >= 2)
                    def _():
                        pltpu.make_async_copy(mbufs.at[k], spacc.at[dst_v.at[b]],
                                              ssem.at[k]).wait()

                    for v in range(8):
                        sl = pl.ds(b * EB + v * 16, 16)
                        srcv = src_v[sl]
                        dstv = dst_v[b, pl.ds(v * 16, 16)]
                        rows = jnp.full((16,), v * 16, jnp.int32) + lax.iota(jnp.int32, 16)
                        for h in range(heads):
                            asrc = _tbl_gather(srcv + h * NPAD)
                            adst = _tbl_gather(dstv + (heads + h) * NPAD)
                            e = asrc + adst
                            e = jnp.where(e >= 0.0, e, 0.2 * e)
                            m = jnp.maximum(gmvec[h] + adst, 0.0)
                            w = jnp.exp(e - m)
                            w_v[h, sl] = w
                            plsc.store_scatter(
                                mbufs.at[k], [rows, jnp.full((16,), h, jnp.int32)], w)
                    pltpu.async_copy(mbufs.at[k], spacc.at[dst_v.at[b]], ssem.at[k],
                                     add=True)

            for k in range(2):
                pltpu.make_async_copy(mbufs.at[k], spacc.at[dst_v.at[0]],
                                      ssem.at[k]).wait()
            plsc.subcore_barrier()
            pltpu.sync_copy(spacc.at[pl.ds(sub * ROWS_PT, ROWS_PT)],
                            den_hbm.at[core].at[pl.ds(sub * ROWS_PT, ROWS_PT)])

            for c in range(nchunks):
                hh = c // cph

                def _mul(b, k, hh=hh):
                    @pl.loop(0, EB // 16)
                    def _(rb):
                        wv = w_v[hh, pl.ds(b * EB + rb * 16, 16)]
                        for l in range(16):
                            ws = wv[l]
                            for j in range(VPR):
                                slj = pl.ds(j * 16, 16)
                                mbufs[k, rb * 16 + l, slj] = (
                                    buf[k, rb * 16 + l, slj] * ws)

                for k in range(2):
                    pltpu.async_copy(xlc_hbm.at[c].at[src_v.at[pl.ds(k * EB, EB)]],
                                     buf.at[k], gsem.at[k])

                @pl.loop(0, EB)
                def _(r):
                    for j in range(VPR):
                        mbufs[0, r, pl.ds(j * 16, 16)] = z16

                for k in range(ROWS_PT // EB):
                    pltpu.async_copy(mbufs.at[0],
                                     spacc.at[pl.ds(sub * ROWS_PT + k * EB, EB)],
                                     zsem)
                for k in range(ROWS_PT // EB):
                    pltpu.make_async_copy(
                        mbufs.at[0], spacc.at[pl.ds(sub * ROWS_PT + k * EB, EB)],
                        zsem).wait()
                plsc.subcore_barrier()

                @pl.loop(0, nb - 2, step=2)
                def _(g):
                    for k in range(2):
                        b = g + k
                        pltpu.make_async_copy(
                            xlc_hbm.at[c].at[src_v.at[pl.ds(b * EB, EB)]],
                            buf.at[k], gsem.at[k]).wait()

                        @pl.when(b >= 2)
                        def _():
                            pltpu.make_async_copy(mbufs.at[k], spacc.at[dst_v.at[b]],
                                                  ssem.at[k]).wait()

                        _mul(b, k)
                        pltpu.async_copy(mbufs.at[k], spacc.at[dst_v.at[b]],
                                         ssem.at[k], add=True)
                        pltpu.async_copy(
                            xlc_hbm.at[c].at[src_v.at[pl.ds((b + 2) * EB, EB)]],
                            buf.at[k], gsem.at[k])

                for k in range(2):
                    b = nb - 2 + k
                    pltpu.make_async_copy(
                        xlc_hbm.at[c].at[src_v.at[pl.ds(b * EB, EB)]],
                        buf.at[k], gsem.at[k]).wait()
                    pltpu.make_async_copy(mbufs.at[k], spacc.at[dst_v.at[b]],
                                          ssem.at[k]).wait()
                    _mul(b, k)
                    pltpu.async_copy(mbufs.at[k], spacc.at[dst_v.at[b]],
                                     ssem.at[k], add=True)
                for k in range(2):
                    pltpu.make_async_copy(mbufs.at[k], spacc.at[dst_v.at[0]],
                                          ssem.at[k]).wait()

                plsc.subcore_barrier()
                pltpu.sync_copy(spacc.at[pl.ds(sub * ROWS_PT, ROWS_PT)],
                                part_hbm.at[core, c].at[pl.ds(sub * ROWS_PT, ROWS_PT)])

        pl.run_scoped(_phases, pltpu.VMEM((tslab, EB, CW), jnp.float32))

    return sc_edge


_sc_edge_l1 = _make_sc_edge(HEADS, NC1)
_sc_edge_l2 = _make_sc_edge(1, NC2)

f32 = jnp.float32


def _run_mm1(x_pad, W1, A1):
    return pl.pallas_call(
        _mm1_body,
        grid=(GRID,),
        in_specs=[
            pl.BlockSpec((RB, D_IN), lambda i: (i, 0)),
            pl.BlockSpec((D_IN, 2 * HID), lambda i: (0, 0)),
            pl.BlockSpec((2 * HID, 128), lambda i: (0, 0)),
        ],
        out_specs=[
            pl.BlockSpec((NC1, RB, CW), lambda i: (0, i, 0)),
            pl.BlockSpec((RB, 128), lambda i: (i, 0)),
            pl.BlockSpec((8, 128), lambda i: (0, 0)),
        ],
        out_shape=[
            jax.ShapeDtypeStruct((NC1, NPAD, CW), f32),
            jax.ShapeDtypeStruct((NPAD, 128), f32),
            jax.ShapeDtypeStruct((8, 128), f32),
        ],
    )(x_pad, W1, A1)


def _run_mm2(parts1, dens1, b1m, W2, A2):
    return pl.pallas_call(
        _mm2_body,
        grid=(GRID,),
        in_specs=[
            pl.BlockSpec((2, NC1, RB, CW), lambda i: (0, 0, i, 0)),
            pl.BlockSpec((2, RB, CW), lambda i: (0, i, 0)),
            pl.BlockSpec((NC1, CW), lambda i: (0, 0)),
            pl.BlockSpec((2 * HID, OUT), lambda i: (0, 0)),
            pl.BlockSpec((OUT, 128), lambda i: (0, 0)),
        ],
        out_specs=[
            pl.BlockSpec((NC2, RB, CW), lambda i: (0, i, 0)),
            pl.BlockSpec((RB, 128), lambda i: (i, 0)),
            pl.BlockSpec((8, 128), lambda i: (0, 0)),
        ],
        out_shape=[
            jax.ShapeDtypeStruct((NC2, NPAD, CW), f32),
            jax.ShapeDtypeStruct((NPAD, 128), f32),
            jax.ShapeDtypeStruct((8, 128), f32),
        ],
    )(parts1, dens1, b1m, W2, A2)


def _run_fin(parts2, dens2, b2m):
    return pl.pallas_call(
        _fin_body,
        grid=(GRID,),
        in_specs=[
            pl.BlockSpec((2, NC2, RB, CW), lambda i: (0, 0, i, 0)),
            pl.BlockSpec((2, RB, CW), lambda i: (0, i, 0)),
            pl.BlockSpec((NC2, CW), lambda i: (0, 0)),
        ],
        out_specs=pl.BlockSpec((RB, OUT), lambda i: (i, 0)),
        out_shape=jax.ShapeDtypeStruct((NPAD, OUT), f32),
    )(parts2, dens2, b2m)


def kernel(x, edge_index, W1, att_src1, att_dst1, b1, W2, att_src2, att_dst2, b2):
    f32 = jnp.float32
    x_pad = jnp.pad(x, ((0, NPAD - N), (0, 0)))
    src = jnp.pad(edge_index[0], (0, EPAD_ARR - E))
    dst = jnp.pad(edge_index[1], (0, EPAD_ARR - E), constant_values=N)
    dst2d = dst.reshape(EPAD_ARR // EB, EB)

    as1 = att_src1.reshape(HEADS, HID)
    ad1 = att_dst1.reshape(HEADS, HID)
    z = jnp.zeros((HID,), f32)
    A1 = jnp.stack([
        jnp.concatenate([as1[0], z]), jnp.concatenate([z, as1[1]]),
        jnp.concatenate([ad1[0], z]), jnp.concatenate([z, ad1[1]]),
    ], axis=1)
    A1 = jnp.pad(A1, ((0, 0), (0, 124)))
    A2 = jnp.stack([att_src2.reshape(OUT), att_dst2.reshape(OUT)], axis=1)
    A2 = jnp.pad(A2, ((0, 0), (0, 126)))

    xlc1, ab1, gmax1 = _run_mm1(x_pad, W1, A1)

    tbl1 = jnp.concatenate([ab1[:, 0], ab1[:, 1], ab1[:, 2], ab1[:, 3]])
    tbl1 = tbl1.reshape(-1, EB, CW)
    gv1 = jnp.pad(gmax1[0, 0:HEADS], (0, 16 - HEADS))

    parts1, dens1 = _sc_edge_l1(xlc1, tbl1, gv1, src, dst2d)

    b1m = b1.reshape(NC1, CW)
    xlc2, ab2, gmax2 = _run_mm2(parts1, dens1, b1m, W2, A2)

    tbl2 = jnp.pad(jnp.concatenate([ab2[:, 0], ab2[:, 1]]),
                   (0, 3 * EB * CW - 2 * NPAD)).reshape(-1, EB, CW)
    gv2 = jnp.pad(gmax2[0, 0:1], (0, 15))

    parts2, dens2 = _sc_edge_l2(xlc2, tbl2, gv2, src, dst2d)

    b2m = b2.reshape(NC2, CW)
    out = _run_fin(parts2, dens2, b2m)

    return out[:N]
